# Initial kernel scaffold; baseline (speedup 1.0000x reference)
import jax
import jax.numpy as jnp
from jax.experimental import pallas as pl


def kernel(f_atoms, f_bonds, a2b, b2a, b2revb, a_scope, W_i_atom, W_i_bond, W_h_0, W_h_1, W_lr, W_o_w, W_o_b, gru_bias, W_ih_f, W_hh_f, b_ih_f, b_hh_f, W_ih_b, W_hh_b, b_ih_b, b_hh_b):
    raise NotImplementedError("write your pallas kernel here")


# speedup vs baseline: 6.1892x; 6.1892x over previous
import functools

import jax
import jax.numpy as jnp
from jax import lax
from jax.experimental import pallas as pl
from jax.experimental.pallas import tpu as pltpu
from jax.experimental.pallas import tpu_sc as plsc

F32 = jnp.float32

H = 128
NA = 50001
NB = 200001
NMOL = 1000
MOLSZ = 50
MAXB = 6
NA_PAD = 51200
NB_PAD = 200704

NC = 2
NS = 16
NW = NC * NS

ATOMS_W = NA_PAD // NW
A_BLK = 64
N_ABLK = ATOMS_W // A_BLK
BONDS_W = NB_PAD // NW
B_BLK = 128
N_BBLK = BONDS_W // B_BLK

ROW_BLK = 2048


def _sc_mesh():
    return plsc.VectorSubcoreMesh(core_axis_name="c", subcore_axis_name="s")


def _sc_a2b_combine(mb, idx2d, base):
    with_base = base is not None
    n_gather = A_BLK * MAXB // 128

    scratch = [
        pltpu.VMEM((A_BLK * MAXB,), jnp.int32),
        pltpu.VMEM((A_BLK * MAXB, H), F32),
        pltpu.VMEM((A_BLK, H), F32),
    ]
    if with_base:
        scratch.append(pltpu.VMEM((A_BLK, H), F32))
    scratch.append(pltpu.SemaphoreType.DMA)

    def body(*refs):
        if with_base:
            (mb_hbm, idx_hbm, base_hbm, out_hbm,
             idx_v, rows_v, out_v, base_v, sem) = refs
        else:
            (mb_hbm, idx_hbm, out_hbm,
             idx_v, rows_v, out_v, sem) = refs
        wid = lax.axis_index("s") * NC + lax.axis_index("c")
        a_base = wid * ATOMS_W

        def blk_body(blk, carry):
            a0 = a_base + blk * A_BLK
            pltpu.sync_copy(idx_hbm.at[pl.ds(a0 * MAXB, A_BLK * MAXB)], idx_v)
            if with_base:
                pltpu.sync_copy(base_hbm.at[pl.ds(a0, A_BLK)], base_v)
            cps = [
                pltpu.async_copy(mb_hbm.at[idx_v.at[pl.ds(j * 128, 128)]],
                                 rows_v.at[pl.ds(j * 128, 128)], sem)
                for j in range(n_gather)
            ]
            for cp in cps:
                cp.wait()

            def atom_body(a, c2):
                for c in range(H // 16):
                    sl = pl.ds(c * 16, 16)
                    v = [rows_v[a * MAXB + j, sl] for j in range(MAXB)]
                    s = ((v[0] + v[4]) + v[2]) + ((v[1] + v[5]) + v[3])
                    m = jnp.maximum(jnp.maximum(jnp.maximum(v[0], v[4]), v[2]),
                                    jnp.maximum(jnp.maximum(v[1], v[5]), v[3]))
                    r = s * m
                    if with_base:
                        r = r + base_v[a, sl]
                    out_v[a, sl] = r
                return c2

            lax.fori_loop(0, A_BLK, atom_body, 0)
            pltpu.sync_copy(out_v, out_hbm.at[pl.ds(a0, A_BLK)])
            return carry

        lax.fori_loop(0, N_ABLK, blk_body, 0)

    kern = pl.kernel(
        body,
        out_type=jax.ShapeDtypeStruct((NA_PAD, H), F32),
        mesh=_sc_mesh(),
        scratch_types=scratch,
    )
    if with_base:
        return kern(mb, idx2d, base)
    return kern(mb, idx2d)


def _sc_bond_diff(ma, mb, b2a_p, b2revb_p):
    def body(ma_hbm, mb_hbm, b2a_hbm, b2revb_hbm, out_hbm,
             aidx_v, ridx_v, arow_v, rrow_v, sem_a, sem_r):
        wid = lax.axis_index("s") * NC + lax.axis_index("c")
        e_base = wid * BONDS_W

        def blk_body(blk, carry):
            e0 = e_base + blk * B_BLK
            pltpu.sync_copy(b2a_hbm.at[pl.ds(e0, B_BLK)], aidx_v)
            pltpu.sync_copy(b2revb_hbm.at[pl.ds(e0, B_BLK)], ridx_v)
            cpa = pltpu.async_copy(ma_hbm.at[aidx_v], arow_v, sem_a)
            cpr = pltpu.async_copy(mb_hbm.at[ridx_v], rrow_v, sem_r)
            cpa.wait()
            cpr.wait()

            def row_body(b, c2):
                for c in range(H // 16):
                    sl = pl.ds(c * 16, 16)
                    arow_v[b, sl] = arow_v[b, sl] - rrow_v[b, sl]
                return c2

            lax.fori_loop(0, B_BLK, row_body, 0)
            pltpu.sync_copy(arow_v, out_hbm.at[pl.ds(e0, B_BLK)])
            return carry

        lax.fori_loop(0, N_BBLK, blk_body, 0)

    kern = pl.kernel(
        body,
        out_type=jax.ShapeDtypeStruct((NB_PAD, H), F32),
        mesh=_sc_mesh(),
        scratch_types=[
            pltpu.VMEM((B_BLK,), jnp.int32),
            pltpu.VMEM((B_BLK,), jnp.int32),
            pltpu.VMEM((B_BLK, H), F32),
            pltpu.VMEM((B_BLK, H), F32),
            pltpu.SemaphoreType.DMA,
            pltpu.SemaphoreType.DMA,
        ],
    )
    return kern(ma, mb, b2a_p, b2revb_p)


def _rowmm(xs, wTs, n_out_rows, adds=(), bias=None, act=False):
    nx = len(xs)
    nadd = len(adds)
    grid = (pl.cdiv(n_out_rows, ROW_BLK),)

    def body(*refs):
        xrefs = refs[:nx]
        wrefs = refs[nx:2 * nx]
        arefs = refs[2 * nx:2 * nx + nadd]
        pos = 2 * nx + nadd
        b_ref = refs[pos] if bias is not None else None
        o_ref = refs[-1]
        acc = jnp.dot(xrefs[0][...], wrefs[0][...],
                      preferred_element_type=F32)
        for i in range(1, nx):
            acc = acc + jnp.dot(xrefs[i][...], wrefs[i][...],
                                preferred_element_type=F32)
        for a_ref in arefs:
            acc = acc + a_ref[...]
        if b_ref is not None:
            acc = acc + b_ref[...]
        if act:
            acc = jnp.maximum(acc, 0.0)
        o_ref[...] = acc

    in_specs = (
        [pl.BlockSpec((ROW_BLK, x.shape[1]), lambda i: (i, 0)) for x in xs]
        + [pl.BlockSpec(wT.shape, lambda i: (0, 0)) for wT in wTs]
        + [pl.BlockSpec((ROW_BLK, H), lambda i: (i, 0)) for _ in adds]
    )
    args = list(xs) + list(wTs) + list(adds)
    if bias is not None:
        in_specs.append(pl.BlockSpec(bias.shape, lambda i: (0, 0)))
        args.append(bias)

    return pl.pallas_call(
        body,
        grid=grid,
        in_specs=in_specs,
        out_specs=pl.BlockSpec((ROW_BLK, H), lambda i: (i, 0)),
        out_shape=jax.ShapeDtypeStruct((n_out_rows, H), F32),
    )(*args)


def _h0_max(hs):
    MB = 40

    def body(h_ref, o_ref):
        m = h_ref[:, 0, :]
        for t in range(1, MOLSZ):
            m = jnp.maximum(m, h_ref[:, t, :])
        o_ref[...] = m

    return pl.pallas_call(
        body,
        grid=(NMOL // MB,),
        in_specs=[pl.BlockSpec((MB, MOLSZ, H), lambda i: (i, 0, 0))],
        out_specs=pl.BlockSpec((MB, H), lambda i: (i, 0)),
        out_shape=jax.ShapeDtypeStruct((NMOL, H), F32),
    )(hs)


def _gru_bidir(hs_t, h0, gbias, wih_f, whh_f, bih_f, bhh_f,
               wih_b, whh_b, bih_b, bhh_b):
    def body(hsf_ref, hsb_ref, h0_ref, gb_ref,
             wihf_ref, whhf_ref, bihf_ref, bhhf_ref,
             wihb_ref, whhb_ref, bihb_ref, bhhb_ref,
             of_ref, ob_ref, hf, hb):
        t = pl.program_id(0)

        @pl.when(t == 0)
        def _():
            hf[...] = h0_ref[...]
            hb[...] = h0_ref[...]

        def cell(x_ref, h_scr, wih, whh, bih, bhh):
            x = jnp.maximum(x_ref[0, :, :] + gb_ref[...], 0.0)
            h = h_scr[...]
            gi = jnp.dot(x, wih[...], preferred_element_type=F32) + bih[...]
            gh = jnp.dot(h, whh[...], preferred_element_type=F32) + bhh[...]
            sig = lambda v: 1.0 / (1.0 + jnp.exp(-v))
            tnh = lambda v: 1.0 - 2.0 / (jnp.exp(2.0 * v) + 1.0)
            r = sig(gi[:, :H] + gh[:, :H])
            z = sig(gi[:, H:2 * H] + gh[:, H:2 * H])
            n = tnh(gi[:, 2 * H:] + r * gh[:, 2 * H:])
            hn = (1.0 - z) * n + z * h
            h_scr[...] = hn
            return hn

        of_ref[0, :, :] = cell(hsf_ref, hf, wihf_ref, whhf_ref,
                               bihf_ref, bhhf_ref)
        ob_ref[0, :, :] = cell(hsb_ref, hb, wihb_ref, whhb_ref,
                               bihb_ref, bhhb_ref)

    full = lambda shape: pl.BlockSpec(shape, lambda t: tuple(0 for _ in shape))
    in_specs = [
        pl.BlockSpec((1, NMOL, H), lambda t: (t, 0, 0)),
        pl.BlockSpec((1, NMOL, H), lambda t: (MOLSZ - 1 - t, 0, 0)),
        full((NMOL, H)),
        full((1, H)),
        full((H, 3 * H)), full((H, 3 * H)), full((1, 3 * H)), full((1, 3 * H)),
        full((H, 3 * H)), full((H, 3 * H)), full((1, 3 * H)), full((1, 3 * H)),
    ]
    out_specs = [
        pl.BlockSpec((1, NMOL, H), lambda t: (t, 0, 0)),
        pl.BlockSpec((1, NMOL, H), lambda t: (MOLSZ - 1 - t, 0, 0)),
    ]
    return pl.pallas_call(
        body,
        grid=(MOLSZ,),
        in_specs=in_specs,
        out_specs=out_specs,
        out_shape=[jax.ShapeDtypeStruct((MOLSZ, NMOL, H), F32)] * 2,
        scratch_shapes=[pltpu.VMEM((NMOL, H), F32),
                        pltpu.VMEM((NMOL, H), F32)],
    )(hs_t, hs_t, h0, gbias, wih_f, whh_f, bih_f, bhh_f,
      wih_b, whh_b, bih_b, bhh_b)


def kernel(f_atoms, f_bonds, a2b, b2a, b2revb, a_scope,
           W_i_atom, W_i_bond, W_h_0, W_h_1, W_lr, W_o_w, W_o_b, gru_bias,
           W_ih_f, W_hh_f, b_ih_f, b_hh_f, W_ih_b, W_hh_b, b_ih_b, b_hh_b):
    a2b_flat = jnp.pad(a2b.astype(jnp.int32).reshape(-1),
                       (0, NA_PAD * MAXB - NA * MAXB))
    b2a_p = jnp.pad(b2a.astype(jnp.int32), (0, NB_PAD - NB))
    b2revb_p = jnp.pad(b2revb.astype(jnp.int32), (0, NB_PAD - NB))

    WiaT = W_i_atom.T
    WibT = W_i_bond.T
    Wh0T = W_h_0.T
    Wh1T = W_h_1.T
    Wl1T = W_lr[:, :H].T
    Wl2T = W_lr[:, H:2 * H].T
    Wl3T = W_lr[:, 2 * H:].T
    Wo1T = W_o_w[:, :H].T
    Wo2T = W_o_w[:, H:].T
    WihT_f = W_ih_f.T
    WhhT_f = W_hh_f.T
    WihT_b = W_ih_b.T
    WhhT_b = W_hh_b.T
    gb = gru_bias.reshape(1, H)
    bihf = b_ih_f.reshape(1, 3 * H)
    bhhf = b_hh_f.reshape(1, 3 * H)
    bihb = b_ih_b.reshape(1, 3 * H)
    bhhb = b_hh_b.reshape(1, 3 * H)
    wob = W_o_b.reshape(1, H)

    ia = _rowmm([f_atoms], [WiaT], NA_PAD, act=True)
    ib = _rowmm([f_bonds], [WibT], NB_PAD, act=True)

    ma = _sc_a2b_combine(ib, a2b_flat, ia)
    diff = _sc_bond_diff(ma, ib, b2a_p, b2revb_p)
    mb = _rowmm([diff], [Wh0T], NB_PAD, adds=[ib], act=True)

    ma = _sc_a2b_combine(mb, a2b_flat, ma)
    diff = _sc_bond_diff(ma, mb, b2a_p, b2revb_p)
    mb = _rowmm([diff], [Wh1T], NB_PAD, adds=[ib], act=True)

    agg = _sc_a2b_combine(mb, a2b_flat, None)

    hidden = _rowmm([agg, ma, ia], [Wl1T, Wl2T, Wl3T], NA_PAD)

    hs = hidden[1:NA].reshape(NMOL, MOLSZ, H)
    h0 = _h0_max(hs)
    hs_t = hs.transpose(1, 0, 2)
    fwd_t, bwd_t = _gru_bidir(hs_t, h0, gb, WihT_f, WhhT_f, bihf, bhhf,
                              WihT_b, WhhT_b, bihb, bhhb)
    fwd = fwd_t.transpose(1, 0, 2).reshape(NMOL * MOLSZ, H)
    bwd = bwd_t.transpose(1, 0, 2).reshape(NMOL * MOLSZ, H)

    m0 = jnp.maximum(hidden[0:1] + gb, 0.0)
    A = jnp.concatenate([m0, fwd], axis=0)
    B = jnp.concatenate([m0, bwd], axis=0)
    return _rowmm([A, B], [Wo1T, Wo2T], NA, bias=wob, act=True)
